# baseline (speedup 1.0000x reference)
import functools

import jax
import jax.numpy as jnp
from jax import lax
from jax.experimental import pallas as pl
from jax.experimental.pallas import tpu as pltpu
from jax.experimental.pallas import tpu_sc as plsc

VOCAB = 1000000
EMBED_DIM = 64
BATCH = 16384

NUM_CORES = 2
NUM_SUBCORES = 16
NUM_WORKERS = NUM_CORES * NUM_SUBCORES
B_PER_W = BATCH // NUM_WORKERS
GROUP = 8
LANES = 16
BKT_CAP = B_PER_W + LANES
POS_BITS = 10
ROW_BYTES = EMBED_DIM * 4

_mesh = plsc.VectorSubcoreMesh(core_axis_name="c", subcore_axis_name="s")


@functools.partial(
    pl.kernel,
    mesh=_mesh,
    out_type=jax.ShapeDtypeStruct((BATCH, EMBED_DIM), jnp.float32),
    scratch_types=[
        pltpu.VMEM((B_PER_W,), jnp.int32),
        pltpu.VMEM((GROUP, BKT_CAP), jnp.int32),
        pltpu.VMEM((B_PER_W + LANES, EMBED_DIM), jnp.float32),
        pltpu.SemaphoreType.DMA,
    ],
    compiler_params=pltpu.CompilerParams(needs_layout_passes=False),
)
def _gather_kernel(table_hbm, idx_hbm, out_hbm, idx_v, bkt, rows_v, gsem):
    wid = lax.axis_index("s") * NUM_CORES + lax.axis_index("c")
    base = wid * B_PER_W
    iota = lax.iota(jnp.int32, LANES)

    pltpu.sync_copy(idx_hbm.at[pl.ds(base, B_PER_W)], idx_v)

    dummy = jnp.full((LANES,), B_PER_W, jnp.int32)
    for k in range(GROUP):
        for g in range(BKT_CAP // LANES):
            bkt[k, pl.ds(g * LANES, LANES)] = dummy

    counts = [jnp.int32(0)] * GROUP
    for g in range(B_PER_W // LANES):
        iv = idx_v[pl.ds(g * LANES, LANES)]
        pv = iota + g * LANES
        packed = lax.bitwise_or(lax.shift_left(iv, POS_BITS), pv)
        rv = lax.bitwise_and(iv, GROUP - 1)
        for k in range(GROUP):
            m = rv == k
            plsc.store_compressed(bkt.at[k, pl.ds(counts[k], LANES)], packed, mask=m)
            counts[k] = counts[k] + plsc.all_reduce_population_count(m)[0]

    n_groups = jnp.int32(0)
    for k in range(GROUP):
        gk = lax.shift_right_logical(counts[k] + (LANES - 1), 4)

        def issue(g, _, k=k):
            wv = bkt[k, pl.ds(pl.multiple_of(g * LANES, LANES), LANES)]
            for i in range(LANES):
                w = wv[i]
                p = lax.bitwise_and(w, (1 << POS_BITS) - 1)
                b8 = lax.shift_left(
                    lax.shift_right_logical(w, POS_BITS + 3), 3
                )
                pltpu.async_copy(
                    table_hbm.at[pl.multiple_of(b8, GROUP) + k],
                    rows_v.at[p],
                    gsem,
                )
            return 0

        lax.fori_loop(0, gk, issue, 0)
        n_groups = n_groups + gk

    def drain(_, __):
        pltpu.make_async_copy(
            table_hbm.at[pl.ds(0, LANES)], rows_v.at[pl.ds(0, LANES)], gsem
        ).wait()
        return 0

    lax.fori_loop(0, n_groups, drain, 0)
    pltpu.sync_copy(rows_v.at[pl.ds(0, B_PER_W)], out_hbm.at[pl.ds(base, B_PER_W)])


ROWS_PER_W = (VOCAB // NUM_WORKERS) // GROUP * GROUP
ROWS_TAIL = VOCAB - ROWS_PER_W * NUM_WORKERS


@functools.partial(
    pl.kernel,
    mesh=_mesh,
    out_type=jax.ShapeDtypeStruct((VOCAB, EMBED_DIM), jnp.float32),
    scratch_types=[pltpu.SemaphoreType.DMA],
    compiler_params=pltpu.CompilerParams(needs_layout_passes=False),
)
def _stage_kernel(table_hbm, dst_hbm, sem):
    wid = lax.axis_index("s") * NUM_CORES + lax.axis_index("c")
    start = pl.multiple_of(wid * ROWS_PER_W, GROUP)
    cp = pltpu.make_async_copy(
        table_hbm.at[pl.ds(start, ROWS_PER_W)],
        dst_hbm.at[pl.ds(start, ROWS_PER_W)],
        sem,
    )
    cp.start()

    @pl.when(wid == 0)
    def _():
        pltpu.async_copy(
            table_hbm.at[pl.ds(VOCAB - ROWS_TAIL, ROWS_TAIL)],
            dst_hbm.at[pl.ds(VOCAB - ROWS_TAIL, ROWS_TAIL)],
            sem,
        )

    cp.wait()

    @pl.when(wid == 0)
    def _():
        pltpu.make_async_copy(
            table_hbm.at[pl.ds(VOCAB - ROWS_TAIL, ROWS_TAIL)],
            dst_hbm.at[pl.ds(VOCAB - ROWS_TAIL, ROWS_TAIL)],
            sem,
        ).wait()


def kernel(embed_cache, indices):
    idx = indices.astype(jnp.int32)
    staged = _stage_kernel(embed_cache)
    return _gather_kernel(staged, idx)


# speedup vs baseline: 58.9319x; 58.9319x over previous
import functools

import jax
import jax.numpy as jnp
from jax import lax
from jax.experimental import pallas as pl
from jax.experimental.pallas import tpu as pltpu
from jax.experimental.pallas import tpu_sc as plsc

VOCAB = 1000000
EMBED_DIM = 64
BATCH = 16384

NUM_CORES = 2
NUM_SUBCORES = 16
NUM_WORKERS = NUM_CORES * NUM_SUBCORES
B_PER_W = BATCH // NUM_WORKERS
GROUP = 8
LANES = 16
BKT_CAP = B_PER_W + LANES
POS_BITS = 10
ROW_BYTES = EMBED_DIM * 4

_mesh = plsc.VectorSubcoreMesh(core_axis_name="c", subcore_axis_name="s")


@functools.partial(
    pl.kernel,
    mesh=_mesh,
    out_type=jax.ShapeDtypeStruct((BATCH, EMBED_DIM), jnp.float32),
    scratch_types=[
        pltpu.VMEM((B_PER_W,), jnp.int32),
        pltpu.VMEM((GROUP, BKT_CAP), jnp.int32),
        pltpu.VMEM((B_PER_W + LANES, EMBED_DIM), jnp.float32),
        pltpu.SemaphoreType.DMA,
    ],
    compiler_params=pltpu.CompilerParams(needs_layout_passes=False),
)
def _gather_kernel(table_hbm, idx_hbm, out_hbm, idx_v, bkt, rows_v, gsem):
    wid = lax.axis_index("s") * NUM_CORES + lax.axis_index("c")
    base = wid * B_PER_W
    iota = lax.iota(jnp.int32, LANES)

    pltpu.sync_copy(idx_hbm.at[pl.ds(base, B_PER_W)], idx_v)

    dummy = jnp.full((LANES,), B_PER_W, jnp.int32)
    for k in range(GROUP):
        for g in range(BKT_CAP // LANES):
            bkt[k, pl.ds(g * LANES, LANES)] = dummy

    counts = [jnp.int32(0)] * GROUP
    for g in range(B_PER_W // LANES):
        iv = idx_v[pl.ds(g * LANES, LANES)]
        pv = iota + g * LANES
        packed = lax.bitwise_or(lax.shift_left(iv, POS_BITS), pv)
        rv = lax.bitwise_and(iv, GROUP - 1)
        for k in range(GROUP):
            m = rv == k
            plsc.store_compressed(bkt.at[k, pl.ds(counts[k], LANES)], packed, mask=m)
            counts[k] = counts[k] + plsc.all_reduce_population_count(m)[0]

    n_groups = jnp.int32(0)
    for k in range(GROUP):
        gk = lax.shift_right_logical(counts[k] + (LANES - 1), 4)

        def issue(g, _, k=k):
            wv = bkt[k, pl.ds(pl.multiple_of(g * LANES, LANES), LANES)]
            for i in range(LANES):
                w = wv[i]
                p = lax.bitwise_and(w, (1 << POS_BITS) - 1)
                sv = lax.shift_right_logical(w, POS_BITS + 3)
                pltpu.async_copy(
                    table_hbm.at[sv, k],
                    rows_v.at[p],
                    gsem,
                )
            return 0

        lax.fori_loop(0, gk, issue, 0)
        n_groups = n_groups + gk

    def drain(_, __):
        pltpu.make_async_copy(
            table_hbm.at[pl.ds(0, LANES), 0], rows_v.at[pl.ds(0, LANES)], gsem
        ).wait()
        return 0

    lax.fori_loop(0, n_groups, drain, 0)
    pltpu.sync_copy(rows_v.at[pl.ds(0, B_PER_W)], out_hbm.at[pl.ds(base, B_PER_W)])


def kernel(embed_cache, indices):
    table3 = embed_cache.reshape(VOCAB // GROUP, GROUP, EMBED_DIM)
    idx = indices.astype(jnp.int32)
    return _gather_kernel(table3, idx)


# speedup vs baseline: 59.0553x; 1.0021x over previous
import functools

import jax
import jax.numpy as jnp
from jax import lax
from jax.experimental import pallas as pl
from jax.experimental.pallas import tpu as pltpu
from jax.experimental.pallas import tpu_sc as plsc

VOCAB = 1000000
EMBED_DIM = 64
BATCH = 16384

NUM_CORES = 2
NUM_SUBCORES = 16
NUM_WORKERS = NUM_CORES * NUM_SUBCORES
B_PER_W = BATCH // NUM_WORKERS
GROUP = 8
LANES = 16
BKT_CAP = B_PER_W + LANES
POS_BITS = 10
ROW_BYTES = EMBED_DIM * 4

_mesh = plsc.VectorSubcoreMesh(core_axis_name="c", subcore_axis_name="s")


@functools.partial(
    pl.kernel,
    mesh=_mesh,
    out_type=jax.ShapeDtypeStruct((BATCH, EMBED_DIM), jnp.float32),
    scratch_types=[
        pltpu.VMEM((GROUP, 2 * B_PER_W // GROUP), jnp.int32),
        pltpu.VMEM((GROUP, BKT_CAP), jnp.int32),
        pltpu.VMEM((B_PER_W + LANES, EMBED_DIM), jnp.float32),
        pltpu.SemaphoreType.DMA,
    ],
    compiler_params=pltpu.CompilerParams(needs_layout_passes=False),
)
def _gather_kernel(table_hbm, idx_hbm, out_hbm, idx_v, bkt, rows_v, gsem):
    wid = lax.axis_index("s") * NUM_CORES + lax.axis_index("c")
    base = wid * B_PER_W
    iota = lax.iota(jnp.int32, LANES)

    pltpu.sync_copy(idx_hbm.at[lax.shift_right_logical(wid, 1)], idx_v)
    half = lax.bitwise_and(wid, 1)

    dummy = jnp.full((LANES,), B_PER_W, jnp.int32)
    for k in range(GROUP):
        for g in range(BKT_CAP // LANES):
            bkt[k, pl.ds(g * LANES, LANES)] = dummy

    counts = [jnp.int32(0)] * GROUP
    for g in range(B_PER_W // LANES):
        flat = g * LANES
        row = half * (B_PER_W // 128) + flat // 128
        iv = idx_v[row, pl.ds(flat % 128, LANES)]
        pv = iota + g * LANES
        packed = lax.bitwise_or(lax.shift_left(iv, POS_BITS), pv)
        rv = lax.bitwise_and(iv, GROUP - 1)
        for k in range(GROUP):
            m = rv == k
            plsc.store_compressed(bkt.at[k, pl.ds(counts[k], LANES)], packed, mask=m)
            counts[k] = counts[k] + plsc.all_reduce_population_count(m)[0]

    n_groups = jnp.int32(0)
    for k in range(GROUP):
        gk = lax.shift_right_logical(counts[k] + (LANES - 1), 4)

        def issue(g, _, k=k):
            wv = bkt[k, pl.ds(pl.multiple_of(g * LANES, LANES), LANES)]
            for i in range(LANES):
                w = wv[i]
                p = lax.bitwise_and(w, (1 << POS_BITS) - 1)
                sv = lax.shift_right_logical(w, POS_BITS + 3)
                pltpu.async_copy(
                    table_hbm.at[sv, k],
                    rows_v.at[p],
                    gsem,
                )
            return 0

        lax.fori_loop(0, gk, issue, 0)
        n_groups = n_groups + gk

    def drain(_, __):
        pltpu.make_async_copy(
            table_hbm.at[pl.ds(0, LANES), 0], rows_v.at[pl.ds(0, LANES)], gsem
        ).wait()
        return 0

    lax.fori_loop(0, n_groups, drain, 0)
    pltpu.sync_copy(rows_v.at[pl.ds(0, B_PER_W)], out_hbm.at[pl.ds(base, B_PER_W)])


def kernel(embed_cache, indices):
    table3 = embed_cache.reshape(VOCAB // GROUP, GROUP, EMBED_DIM)
    idx = indices.astype(jnp.int32).reshape(BATCH // 1024, GROUP, 128)
    return _gather_kernel(table3, idx)
